# Initial kernel scaffold; baseline (speedup 1.0000x reference)
import jax
import jax.numpy as jnp
from jax.experimental import pallas as pl


def kernel(node_input, batch, mean_shift, affine_weight, affine_bias):
    raise NotImplementedError("write your pallas kernel here")


# speedup vs baseline: 7.8243x; 7.8243x over previous
import numpy as np
import jax
import jax.numpy as jnp
from jax import lax
from jax.experimental import pallas as pl
from jax.experimental.pallas import tpu as pltpu

_G = 512
_DIM = 480
_NCH = 224
_EPS = 1e-5

_ch_of_col = np.empty((_DIM,), np.int64)
_ch_of_col[:128] = np.arange(128)
_ch_of_col[128:320] = 128 + np.arange(192) // 3
_ch_of_col[320:480] = 192 + np.arange(160) // 5
_E_np = (np.arange(_NCH)[:, None] == _ch_of_col[None, :]).astype(np.float32)
_D_np = np.array([1.0] * 128 + [3.0] * 64 + [5.0] * 32, np.float32)[None, :]


def _stats_kernel(batch_ref, x_ref, et_ref, e_ref, ms_ref, ms480_ref, d_ref,
                  w_ref, meanms_ref, norm_ref, s_acc, q_acc, c_acc):
    i = pl.program_id(0)
    nb = pl.num_programs(0)

    @pl.when(i == 0)
    def _init():
        s_acc[...] = jnp.zeros_like(s_acc)
        q_acc[...] = jnp.zeros_like(q_acc)
        c_acc[...] = jnp.zeros_like(c_acc)

    x = x_ref[...]
    bb = batch_ref[0, 0, :]
    onehot_t = (lax.broadcasted_iota(jnp.int32, (_G, x.shape[0]), 0)
                == bb[None, :]).astype(jnp.float32)
    s_acc[...] += jax.lax.dot(onehot_t, x, preferred_element_type=jnp.float32)
    q_acc[...] += jax.lax.dot(onehot_t, x * x,
                              preferred_element_type=jnp.float32)
    c_acc[...] = c_acc[...] + jnp.sum(onehot_t, axis=1, keepdims=True)

    @pl.when(i == nb - 1)
    def _finalize():
        hi = jax.lax.Precision.HIGHEST
        nc = jnp.maximum(c_acc[:, 0:1], 1.0)
        mean = s_acc[...] / nc
        m2 = jax.lax.dot(mean * mean, et_ref[...], precision=hi)
        q224 = jax.lax.dot(q_acc[...], et_ref[...], precision=hi)
        ms = ms_ref[0:1, :]
        var = (q224 + (ms * ms - 2.0 * ms) * m2 * nc) / (nc * d_ref[0:1, :])
        norm224 = lax.rsqrt(var + _EPS) * w_ref[0:1, :]
        norm_ref[...] = jax.lax.dot(norm224, e_ref[...], precision=hi)
        meanms_ref[...] = mean * ms480_ref[0:1, :]


def _apply_kernel(batch_ref, x_ref, meanms_ref, norm_ref, bias_ref, out_ref):
    x = x_ref[...]
    bb = batch_ref[0, 0, :]
    onehot = (lax.broadcasted_iota(jnp.int32, (x.shape[0], _G), 1)
              == bb[:, None]).astype(jnp.float32)
    mm = jax.lax.dot(onehot, meanms_ref[...],
                     preferred_element_type=jnp.float32)
    nn = jax.lax.dot(onehot, norm_ref[...],
                     preferred_element_type=jnp.float32)
    out_ref[...] = (x - mm) * nn + bias_ref[0:1, :]


def kernel(node_input, batch, mean_shift, affine_weight, affine_bias):
    n = node_input.shape[0]
    bsz = 1000
    if n % bsz:
        pad = bsz - n % bsz
        node_input = jnp.pad(node_input, ((0, pad), (0, 0)))
        batch = jnp.pad(batch, (0, pad), constant_values=_G)
    npad = node_input.shape[0]
    nb = npad // bsz

    batch3 = batch.reshape(nb, 1, bsz)
    et = jnp.asarray(_E_np.T)
    e = jnp.asarray(_E_np)
    ms224 = jnp.tile(mean_shift.reshape(1, _NCH), (8, 1))
    ms480 = jnp.tile(mean_shift.reshape(1, _NCH)[:, _ch_of_col], (8, 1))
    dvec = jnp.tile(jnp.asarray(_D_np), (8, 1))
    w = jnp.tile(affine_weight.reshape(1, _NCH), (8, 1))
    nsc = affine_bias.shape[1]
    bias480 = jnp.tile(
        jnp.concatenate(
            [affine_bias.reshape(1, nsc),
             jnp.zeros((1, _DIM - nsc), jnp.float32)], axis=1), (8, 1))

    full = lambda shape: pl.BlockSpec(shape, lambda i: (0,) * len(shape))
    meanms, norm = pl.pallas_call(
        _stats_kernel,
        grid=(nb,),
        in_specs=[
            pl.BlockSpec((1, 1, bsz), lambda i: (i, 0, 0)),
            pl.BlockSpec((bsz, _DIM), lambda i: (i, 0)),
            full((_DIM, _NCH)),
            full((_NCH, _DIM)),
            full((8, _NCH)),
            full((8, _DIM)),
            full((8, _NCH)),
            full((8, _NCH)),
        ],
        out_specs=[full((_G, _DIM)), full((_G, _DIM))],
        out_shape=[
            jax.ShapeDtypeStruct((_G, _DIM), jnp.float32),
            jax.ShapeDtypeStruct((_G, _DIM), jnp.float32),
        ],
        scratch_shapes=[
            pltpu.VMEM((_G, _DIM), jnp.float32),
            pltpu.VMEM((_G, _DIM), jnp.float32),
            pltpu.VMEM((_G, 128), jnp.float32),
        ],
        compiler_params=pltpu.CompilerParams(
            dimension_semantics=("arbitrary",)),
    )(batch3, node_input, et, e, ms224, ms480, dvec, w)

    out = pl.pallas_call(
        _apply_kernel,
        grid=(nb,),
        in_specs=[
            pl.BlockSpec((1, 1, bsz), lambda i: (i, 0, 0)),
            pl.BlockSpec((bsz, _DIM), lambda i: (i, 0)),
            full((_G, _DIM)),
            full((_G, _DIM)),
            full((8, _DIM)),
        ],
        out_specs=pl.BlockSpec((bsz, _DIM), lambda i: (i, 0)),
        out_shape=jax.ShapeDtypeStruct((npad, _DIM), jnp.float32),
        compiler_params=pltpu.CompilerParams(
            dimension_semantics=("arbitrary",)),
    )(batch3, node_input, meanms, norm, bias480)
    return out[:n]


# speedup vs baseline: 8.4138x; 1.0753x over previous
import numpy as np
import jax
import jax.numpy as jnp
from jax import lax
from jax.experimental import pallas as pl
from jax.experimental.pallas import tpu as pltpu

_G = 512
_DIM = 480
_NCH = 224
_EPS = 1e-5

_ch_of_col = np.empty((_DIM,), np.int64)
_ch_of_col[:128] = np.arange(128)
_ch_of_col[128:320] = 128 + np.arange(192) // 3
_ch_of_col[320:480] = 192 + np.arange(160) // 5
_E_np = (np.arange(_NCH)[:, None] == _ch_of_col[None, :]).astype(np.float32)
_D_np = np.array([1.0] * 128 + [3.0] * 64 + [5.0] * 32, np.float32)[None, :]


def _stats_kernel(batch_ref, x_ref, et_ref, e_ref, ms_ref, ms480_ref, d_ref,
                  w_ref, meanms_ref, norm_ref, s_acc, q_acc, c_acc):
    i = pl.program_id(0)
    nb = pl.num_programs(0)

    @pl.when(i == 0)
    def _init():
        s_acc[...] = jnp.zeros_like(s_acc)
        q_acc[...] = jnp.zeros_like(q_acc)
        c_acc[...] = jnp.zeros_like(c_acc)

    x = x_ref[...]
    bb = batch_ref[0, 0, :]
    onehot_t = (lax.broadcasted_iota(jnp.int32, (_G, x.shape[0]), 0)
                == bb[None, :]).astype(jnp.bfloat16)
    xb = x.astype(jnp.bfloat16)
    s_acc[...] += jax.lax.dot(onehot_t, xb, preferred_element_type=jnp.float32)
    q_acc[...] += jax.lax.dot(onehot_t, (x * x).astype(jnp.bfloat16),
                              preferred_element_type=jnp.float32)
    c_acc[...] = c_acc[...] + jnp.sum(onehot_t.astype(jnp.float32), axis=1,
                                      keepdims=True)

    @pl.when(i == nb - 1)
    def _finalize():
        hi = jax.lax.Precision.HIGHEST
        nc = jnp.maximum(c_acc[:, 0:1], 1.0)
        mean = s_acc[...] / nc
        m2 = jax.lax.dot(mean * mean, et_ref[...], precision=hi)
        q224 = jax.lax.dot(q_acc[...], et_ref[...], precision=hi)
        ms = ms_ref[0:1, :]
        var = (q224 + (ms * ms - 2.0 * ms) * m2 * nc) / (nc * d_ref[0:1, :])
        norm224 = lax.rsqrt(var + _EPS) * w_ref[0:1, :]
        norm_ref[...] = jax.lax.dot(norm224, e_ref[...], precision=hi)
        meanms_ref[...] = mean * ms480_ref[0:1, :]


def _apply_kernel(batch_ref, x_ref, meanms_ref, norm_ref, bias_ref, out_ref):
    x = x_ref[...]
    bb = batch_ref[0, 0, :]
    onehot = (lax.broadcasted_iota(jnp.int32, (x.shape[0], _G), 1)
              == bb[:, None]).astype(jnp.bfloat16)
    mm = jax.lax.dot(onehot, meanms_ref[...].astype(jnp.bfloat16),
                     preferred_element_type=jnp.float32)
    nn = jax.lax.dot(onehot, norm_ref[...].astype(jnp.bfloat16),
                     preferred_element_type=jnp.float32)
    out_ref[...] = (x - mm) * nn + bias_ref[0:1, :]


def kernel(node_input, batch, mean_shift, affine_weight, affine_bias):
    n = node_input.shape[0]
    bsz = 2000
    if n % bsz:
        pad = bsz - n % bsz
        node_input = jnp.pad(node_input, ((0, pad), (0, 0)))
        batch = jnp.pad(batch, (0, pad), constant_values=_G)
    npad = node_input.shape[0]
    nb = npad // bsz

    batch3 = batch.reshape(nb, 1, bsz)
    et = jnp.asarray(_E_np.T)
    e = jnp.asarray(_E_np)
    ms224 = jnp.tile(mean_shift.reshape(1, _NCH), (8, 1))
    ms480 = jnp.tile(mean_shift.reshape(1, _NCH)[:, _ch_of_col], (8, 1))
    dvec = jnp.tile(jnp.asarray(_D_np), (8, 1))
    w = jnp.tile(affine_weight.reshape(1, _NCH), (8, 1))
    nsc = affine_bias.shape[1]
    bias480 = jnp.tile(
        jnp.concatenate(
            [affine_bias.reshape(1, nsc),
             jnp.zeros((1, _DIM - nsc), jnp.float32)], axis=1), (8, 1))

    full = lambda shape: pl.BlockSpec(shape, lambda i: (0,) * len(shape))
    meanms, norm = pl.pallas_call(
        _stats_kernel,
        grid=(nb,),
        in_specs=[
            pl.BlockSpec((1, 1, bsz), lambda i: (i, 0, 0)),
            pl.BlockSpec((bsz, _DIM), lambda i: (i, 0)),
            full((_DIM, _NCH)),
            full((_NCH, _DIM)),
            full((8, _NCH)),
            full((8, _DIM)),
            full((8, _NCH)),
            full((8, _NCH)),
        ],
        out_specs=[full((_G, _DIM)), full((_G, _DIM))],
        out_shape=[
            jax.ShapeDtypeStruct((_G, _DIM), jnp.float32),
            jax.ShapeDtypeStruct((_G, _DIM), jnp.float32),
        ],
        scratch_shapes=[
            pltpu.VMEM((_G, _DIM), jnp.float32),
            pltpu.VMEM((_G, _DIM), jnp.float32),
            pltpu.VMEM((_G, 128), jnp.float32),
        ],
        compiler_params=pltpu.CompilerParams(
            dimension_semantics=("arbitrary",)),
    )(batch3, node_input, et, e, ms224, ms480, dvec, w)

    out = pl.pallas_call(
        _apply_kernel,
        grid=(nb,),
        in_specs=[
            pl.BlockSpec((1, 1, bsz), lambda i: (i, 0, 0)),
            pl.BlockSpec((bsz, _DIM), lambda i: (i, 0)),
            full((_G, _DIM)),
            full((_G, _DIM)),
            full((8, _DIM)),
        ],
        out_specs=pl.BlockSpec((bsz, _DIM), lambda i: (i, 0)),
        out_shape=jax.ShapeDtypeStruct((npad, _DIM), jnp.float32),
        compiler_params=pltpu.CompilerParams(
            dimension_semantics=("arbitrary",)),
    )(batch3, node_input, meanms, norm, bias480)
    return out[:n]
